# baseline (speedup 1.0000x reference)
import functools

import jax
import jax.numpy as jnp
from jax import lax
from jax.experimental import pallas as pl
from jax.experimental.pallas import tpu as pltpu

_W = 128
_Q = 2048
_N = 128 * 128
_QB = 256


def _nn_body(q_ref, kt_ref, conf_ref, match_ref, orig_ref, confo_ref):
    q = q_ref[...]
    kt = kt_ref[0]
    q2 = jnp.sum(q * q, axis=1, keepdims=True)
    k2 = jnp.sum(kt * kt, axis=0, keepdims=True)
    cross = jax.lax.dot_general(
        q, kt, (((1,), (0,)), ((), ())),
        preferred_element_type=jnp.float32)
    dist = (q2 + k2) - 2.0 * cross

    min_val = jnp.min(dist, axis=1, keepdims=True)
    iota = lax.broadcasted_iota(jnp.int32, dist.shape, 1)
    min_idx = jnp.min(jnp.where(dist == min_val, iota, _N),
                      axis=1, keepdims=True)
    onehot = (iota == min_idx).astype(jnp.float32)

    orig = jax.lax.dot_general(
        onehot, kt, (((1,), (1,)), ((), ())),
        preferred_element_type=jnp.float32)
    cg = jax.lax.dot_general(
        onehot, conf_ref[0], (((1,), (1,)), ((), ())),
        preferred_element_type=jnp.float32)

    ys = min_idx // _W
    xs = min_idx % _W
    match_ref[0] = jnp.concatenate([ys, xs], axis=1)
    orig_ref[0] = orig
    confo_ref[0] = cg


def _nn_search(q, keys_t, conf_r):
    n_qb = _Q // _QB
    grid = (7, n_qb)
    out_shapes = (
        jax.ShapeDtypeStruct((7, _Q, 2), jnp.int32),
        jax.ShapeDtypeStruct((7, _Q, 3), jnp.float32),
        jax.ShapeDtypeStruct((7, _Q, 1), jnp.float32),
    )
    in_specs = [
        pl.BlockSpec((_QB, 3), lambda s, b: (b, 0)),
        pl.BlockSpec((1, 3, _N), lambda s, b: (s, 0, 0)),
        pl.BlockSpec((1, 1, _N), lambda s, b: (s, 0, 0)),
    ]
    out_specs = (
        pl.BlockSpec((1, _QB, 2), lambda s, b: (s, b, 0)),
        pl.BlockSpec((1, _QB, 3), lambda s, b: (s, b, 0)),
        pl.BlockSpec((1, _QB, 1), lambda s, b: (s, b, 0)),
    )
    return pl.pallas_call(
        _nn_body,
        grid=grid,
        in_specs=in_specs,
        out_specs=out_specs,
        out_shape=out_shapes,
    )(q, keys_t, conf_r)


def kernel(pts, conf):
    S, H, W, _ = pts.shape
    conf0 = conf[0].reshape(-1)
    _, top_idx = jax.lax.top_k(conf0, _Q)
    y0 = top_idx // W
    x0 = top_idx % W
    ref_pts = pts[0].reshape(-1, 3)[top_idx]

    keys_t = pts[1:].reshape(7, _N, 3).transpose(0, 2, 1)
    conf_r = conf[1:].reshape(7, 1, _N)

    matches, orig, confs = _nn_search(ref_pts, keys_t, conf_r)

    m0 = jnp.stack([y0, x0], axis=1)[None]
    track_matches = jnp.concatenate([m0, matches], axis=0)
    original_pts = jnp.concatenate([ref_pts[None], orig], axis=0)
    track_confs = jnp.concatenate([conf0[top_idx][None], confs[..., 0]],
                                  axis=0)
    return track_matches, original_pts, track_confs


# speedup vs baseline: 2.4270x; 2.4270x over previous
import functools

import jax
import jax.numpy as jnp
from jax import lax
from jax.experimental import pallas as pl
from jax.experimental.pallas import tpu as pltpu
from jax.experimental.pallas import tpu_sc as plsc

_W = 128
_Q = 2048
_N = 128 * 128
_QB = 256
_NV = 8

_SC_PARTS = 4
_SC_CHUNK = _Q // _SC_PARTS


def _nn_body(q_ref, kt_ref, idx_ref):
    q = q_ref[...]
    kt = kt_ref[0]
    q2 = jnp.sum(q * q, axis=1, keepdims=True)
    k2 = jnp.sum(kt * kt, axis=0, keepdims=True)
    cross = jax.lax.dot_general(
        q, kt, (((1,), (0,)), ((), ())),
        preferred_element_type=jnp.float32)
    dist = (q2 + k2) - 2.0 * cross

    min_val = jnp.min(dist, axis=1, keepdims=True)
    iota = lax.broadcasted_iota(jnp.int32, dist.shape, 1)
    idx_ref[0] = jnp.min(jnp.where(dist == min_val, iota, _N),
                         axis=1, keepdims=True)


def _nn_search(q, keys_t):
    n_qb = _Q // _QB
    return pl.pallas_call(
        _nn_body,
        grid=(7, n_qb),
        in_specs=[
            pl.BlockSpec((_QB, 3), lambda s, b: (b, 0)),
            pl.BlockSpec((1, 3, _N), lambda s, b: (s, 0, 0)),
        ],
        out_specs=pl.BlockSpec((1, _QB, 1), lambda s, b: (s, b, 0)),
        out_shape=jax.ShapeDtypeStruct((7, _Q, 1), jnp.int32),
    )(q, keys_t)


def _sc_gather_body(idx_hbm, kx_hbm, ky_hbm, kz_hbm, cf_hbm,
                    ys_o, xs_o, px_o, py_o, pz_o, cg_o,
                    idx_v, kx_v, ky_v, kz_v, cf_v,
                    ys_s, xs_s, px_s, py_s, pz_s, cg_s):
    cid = lax.axis_index("c")
    sid = lax.axis_index("s")
    wid = sid * 2 + cid
    view = wid // _SC_PARTS
    base = view * _Q + (wid % _SC_PARTS) * _SC_CHUNK
    tab = view * _N

    pltpu.sync_copy(kx_hbm.at[pl.ds(tab, _N)], kx_v)
    pltpu.sync_copy(ky_hbm.at[pl.ds(tab, _N)], ky_v)
    pltpu.sync_copy(kz_hbm.at[pl.ds(tab, _N)], kz_v)
    pltpu.sync_copy(cf_hbm.at[pl.ds(tab, _N)], cf_v)
    pltpu.sync_copy(idx_hbm.at[pl.ds(base, _SC_CHUNK)], idx_v)

    for i in range(_SC_CHUNK // 16):
        sl = pl.ds(i * 16, 16)
        iv = idx_v[sl]
        px_s[sl] = plsc.load_gather(kx_v, [iv])
        py_s[sl] = plsc.load_gather(ky_v, [iv])
        pz_s[sl] = plsc.load_gather(kz_v, [iv])
        cg_s[sl] = plsc.load_gather(cf_v, [iv])
        ys_s[sl] = iv // _W
        xs_s[sl] = iv % _W

    out_sl = pl.ds(base, _SC_CHUNK)
    pltpu.sync_copy(ys_s, ys_o.at[out_sl])
    pltpu.sync_copy(xs_s, xs_o.at[out_sl])
    pltpu.sync_copy(px_s, px_o.at[out_sl])
    pltpu.sync_copy(py_s, py_o.at[out_sl])
    pltpu.sync_copy(pz_s, pz_o.at[out_sl])
    pltpu.sync_copy(cg_s, cg_o.at[out_sl])


def _sc_gather(all_idx, kx8, ky8, kz8, cf8):
    f32 = jnp.float32
    i32 = jnp.int32
    mesh = plsc.VectorSubcoreMesh(core_axis_name="c", subcore_axis_name="s")
    out_type = (
        jax.ShapeDtypeStruct((_NV * _Q,), i32),
        jax.ShapeDtypeStruct((_NV * _Q,), i32),
        jax.ShapeDtypeStruct((_NV * _Q,), f32),
        jax.ShapeDtypeStruct((_NV * _Q,), f32),
        jax.ShapeDtypeStruct((_NV * _Q,), f32),
        jax.ShapeDtypeStruct((_NV * _Q,), f32),
    )
    scratch_types = [
        pltpu.VMEM((_SC_CHUNK,), i32),
        pltpu.VMEM((_N,), f32),
        pltpu.VMEM((_N,), f32),
        pltpu.VMEM((_N,), f32),
        pltpu.VMEM((_N,), f32),
        pltpu.VMEM((_SC_CHUNK,), i32),
        pltpu.VMEM((_SC_CHUNK,), i32),
        pltpu.VMEM((_SC_CHUNK,), f32),
        pltpu.VMEM((_SC_CHUNK,), f32),
        pltpu.VMEM((_SC_CHUNK,), f32),
        pltpu.VMEM((_SC_CHUNK,), f32),
    ]
    fn = pl.kernel(
        _sc_gather_body,
        mesh=mesh,
        out_type=out_type,
        scratch_types=scratch_types,
        compiler_params=pltpu.CompilerParams(needs_layout_passes=False),
    )
    return fn(all_idx, kx8, ky8, kz8, cf8)


def kernel(pts, conf):
    S, H, W, _ = pts.shape
    conf0 = conf[0].reshape(-1)
    _, top_idx = jax.lax.top_k(conf0, _Q)
    ref_pts = pts[0].reshape(-1, 3)[top_idx]

    keys_t8 = pts.reshape(_NV, _N, 3).transpose(0, 2, 1)
    conf8 = conf.reshape(_NV, _N)

    min_idx = _nn_search(ref_pts, keys_t8[1:])
    all_idx = jnp.concatenate(
        [top_idx[None], min_idx[..., 0]], axis=0).reshape(-1)

    kx8 = keys_t8[:, 0].reshape(-1)
    ky8 = keys_t8[:, 1].reshape(-1)
    kz8 = keys_t8[:, 2].reshape(-1)
    cf8 = conf8.reshape(-1)
    ys, xs, px, py, pz, cg = _sc_gather(all_idx, kx8, ky8, kz8, cf8)

    ys = ys.reshape(_NV, _Q)
    xs = xs.reshape(_NV, _Q)
    track_matches = jnp.stack([ys, xs], axis=-1)
    original_pts = jnp.stack(
        [px.reshape(_NV, _Q), py.reshape(_NV, _Q), pz.reshape(_NV, _Q)],
        axis=-1)
    return track_matches, original_pts, cg.reshape(_NV, _Q)
